# Initial kernel scaffold; baseline (speedup 1.0000x reference)
import jax
import jax.numpy as jnp
from jax.experimental import pallas as pl


def kernel(energy_x, comm_x, tau, tau_max, lambda_min_0, We, be, Wc, bc, msgW, msgB, updW, updB, ln_g, ln_b, dec1W, dec1b, dec2W, dec2b, K):
    raise NotImplementedError("write your pallas kernel here")


# speedup vs baseline: 64.1754x; 64.1754x over previous
import functools

import jax
import jax.numpy as jnp
from jax.experimental import pallas as pl

_H = 8
_INV_SQRT2 = 0.7071067811865476


def _gelu(x):
    return 0.5 * x * (1.0 + jax.lax.erf(x * _INV_SQRT2))


def _gnn_body(n, B, exc, exp, exn, cxc, cxp, cxn,
              WeP, beP, WcP, bcP,
              UEE1, UEN1, UEC1, cE1, VE1, cbE1,
              UCC1, UCN1, UCE1, cC1, VC1, cbC1,
              g1, b1,
              UEE2, UEN2, UEC2, cE2, VE2, cbE2,
              g2, b2,
              psum_ref):
    i = pl.program_id(0)
    H = _H
    BH = B + 2 * H
    f32 = jnp.float32
    D = WeP.shape[1]

    ex = jnp.concatenate([exp[...], exc[...], exn[...]], axis=0)
    cx = jnp.concatenate([cxp[...], cxc[...], cxn[...]], axis=0)

    hE = jnp.dot(ex, WeP[...], preferred_element_type=f32) + beP[...]
    hC = jnp.dot(cx, WcP[...], preferred_element_type=f32) + bcP[...]

    g = (i * B - H) + jax.lax.broadcasted_iota(jnp.int32, (BH, 1), 0)
    m_prev = (g > 0).astype(f32)
    m_next = (g < n - 1).astype(f32)
    zrow = jnp.zeros((1, D), f32)

    def nbr(x):
        down = jnp.concatenate([zrow, x[:-1]], axis=0)
        up = jnp.concatenate([x[1:], zrow], axis=0)
        return m_prev * down + m_next * up

    def ln_gelu_res(x, y, gam, bet):
        mu = jnp.mean(y, axis=-1, keepdims=True)
        var = jnp.mean((y - mu) ** 2, axis=-1, keepdims=True)
        yn = (y - mu) * jax.lax.rsqrt(var + 1e-5) * gam[...] + bet[...]
        return x + _gelu(yn)

    def corrections(y, xs, xo, nb, UN, UC, V, cb):
        UNv, UCv, Vv, cbv = UN[...], UC[...], V[...], cb[...]

        def corr_slice(lo):
            sl = slice(lo, lo + H)
            mask = ((g[sl] == 0) | (g[sl] == n - 1)).astype(f32)
            c = (jnp.dot(nb[sl], UNv, preferred_element_type=f32) * 0.5
                 + jnp.dot(xs[sl], Vv, preferred_element_type=f32)
                 + jnp.dot(xo[sl], UCv, preferred_element_type=f32) * 0.5
                 + cbv)
            return c * mask

        cA = corr_slice(H)
        cB = corr_slice(B)
        return jnp.concatenate(
            [y[:H], y[H:2 * H] + cA, y[2 * H:B], y[B:B + H] + cB, y[B + H:]],
            axis=0)

    nE = nbr(hE)
    nC = nbr(hC)
    yE = (jnp.dot(hE, UEE1[...], preferred_element_type=f32)
          + jnp.dot(nE, UEN1[...], preferred_element_type=f32)
          + jnp.dot(hC, UEC1[...], preferred_element_type=f32) + cE1[...])
    yE = corrections(yE, hE, hC, nE, UEN1, UEC1, VE1, cbE1)
    yC = (jnp.dot(hC, UCC1[...], preferred_element_type=f32)
          + jnp.dot(nC, UCN1[...], preferred_element_type=f32)
          + jnp.dot(hE, UCE1[...], preferred_element_type=f32) + cC1[...])
    yC = corrections(yC, hC, hE, nC, UCN1, UCE1, VC1, cbC1)
    x1E = ln_gelu_res(hE, yE, g1, b1)
    x1C = ln_gelu_res(hC, yC, g1, b1)

    n2 = nbr(x1E)
    y2 = (jnp.dot(x1E, UEE2[...], preferred_element_type=f32)
          + jnp.dot(n2, UEN2[...], preferred_element_type=f32)
          + jnp.dot(x1C, UEC2[...], preferred_element_type=f32) + cE2[...])
    y2 = corrections(y2, x1E, x1C, n2, UEN2, UEC2, VE2, cbE2)
    x2E = ln_gelu_res(x1E, y2, g2, b2)

    psum_ref[...] = jnp.sum(x2E[H:H + B], axis=0, keepdims=True).reshape(1, 1, D)


def _decode_body(n, psums, d1W, d1b, d2W, d2b, tau, tmax, lam, Kv,
                 u_ref, rho_ref):
    f32 = jnp.float32
    G = psums.shape[0]
    mean = jnp.sum(psums[...].reshape(G, -1), axis=0, keepdims=True) * (1.0 / n)
    t = _gelu(jnp.dot(mean, d1W[...], preferred_element_type=f32) + d1b[...])
    u_ref[...] = jnp.dot(t, d2W[...], preferred_element_type=f32) + d2b[...]
    delay = jnp.sum(Kv[...] * (tau[...] / tmax[...]), axis=-1, keepdims=True)
    rho_ref[...] = jnp.abs(lam[...]) - delay


def _pick_block(n):
    best = 8
    for d in range(16, 2501, 8):
        if n % d == 0:
            best = d
    return best


def _fold_layer(msgW_l, msgB_l, updW_l, updB_l):
    D = updW_l.shape[1]
    A0, B0 = msgW_l[0, :D], msgW_l[0, D:]
    A1, B1 = msgW_l[1, :D], msgW_l[1, D:]
    A2, B2 = msgW_l[2, :D], msgW_l[2, D:]
    b0, b1, b2 = msgB_l[0], msgB_l[1], msgB_l[2]
    Wu1, Wu2 = updW_l[:D], updW_l[D:]
    W3 = Wu2 / 3.0
    W6 = Wu2 / 6.0
    r1 = lambda v: v.reshape(1, -1)
    fE = dict(UEE=Wu1 + (2.0 * B0 + B2) @ W3, UEN=A0 @ W3, UEC=A2 @ W3,
              c=r1((2.0 * b0 + b2) @ W3 + updB_l),
              V=(B2 - B0) @ W6, cb=r1((b2 - b0) @ W6))
    fC = dict(UCC=Wu1 + (2.0 * B1 + B2) @ W3, UCN=A1 @ W3, UCE=A2 @ W3,
              c=r1((2.0 * b1 + b2) @ W3 + updB_l),
              V=(B2 - B1) @ W6, cb=r1((b2 - b1) @ W6))
    return fE, fC


def kernel(energy_x, comm_x, tau, tau_max, lambda_min_0, We, be, Wc, bc,
           msgW, msgB, updW, updB, ln_g, ln_b, dec1W, dec1b, dec2W, dec2b, K):
    f32 = jnp.float32
    batch, n = energy_x.shape[0], energy_x.shape[1]
    De, Dc = energy_x.shape[2], comm_x.shape[2]
    D = We.shape[1]
    H = _H
    B = _pick_block(n)
    G = n // B
    nB8 = B // 8
    last8 = n // 8 - 1

    WeP = jnp.pad(We, ((0, 8 - De), (0, 0)))
    WcP = jnp.pad(Wc, ((0, 8 - Dc), (0, 0)))
    exP = jnp.pad(energy_x, ((0, 0), (0, 0), (0, 8 - De)))
    cxP = jnp.pad(comm_x, ((0, 0), (0, 0), (0, 8 - Dc)))
    r1 = lambda v: v.reshape(1, -1)

    fE1, fC1 = _fold_layer(msgW[0], msgB[0], updW[0], updB[0])
    fE2, _ = _fold_layer(msgW[1], msgB[1], updW[1], updB[1])

    idx_cur = lambda i: (i, 0)
    idx_prev = lambda i: (jnp.maximum(i * nB8 - 1, 0), 0)
    idx_next = lambda i: (jnp.minimum((i + 1) * nB8, last8), 0)
    dspec = [pl.BlockSpec((B, 8), idx_cur),
             pl.BlockSpec((8, 8), idx_prev),
             pl.BlockSpec((8, 8), idx_next)]
    wspec = lambda a: pl.BlockSpec(a.shape, lambda i: (0, 0))

    weights = [WeP, r1(be), WcP, r1(bc),
               fE1["UEE"], fE1["UEN"], fE1["UEC"], fE1["c"], fE1["V"], fE1["cb"],
               fC1["UCC"], fC1["UCN"], fC1["UCE"], fC1["c"], fC1["V"], fC1["cb"],
               r1(ln_g[0]), r1(ln_b[0]),
               fE2["UEE"], fE2["UEN"], fE2["UEC"], fE2["c"], fE2["V"], fE2["cb"],
               r1(ln_g[1]), r1(ln_b[1])]

    gnn = pl.pallas_call(
        functools.partial(_gnn_body, n, B),
        grid=(G,),
        in_specs=dspec + dspec + [wspec(w) for w in weights],
        out_specs=pl.BlockSpec((1, 1, D), lambda i: (i, 0, 0)),
        out_shape=jax.ShapeDtypeStruct((G, 1, D), f32),
    )

    decode = pl.pallas_call(
        functools.partial(_decode_body, n),
        out_shape=(jax.ShapeDtypeStruct((1, 2 * dec2W.shape[1] // 2), f32),
                   jax.ShapeDtypeStruct((1, 1), f32)),
    )

    us = []
    rho = None
    for bi in range(batch):
        ex_b = exP[bi]
        cx_b = cxP[bi]
        psums = gnn(ex_b, ex_b, ex_b, cx_b, cx_b, cx_b, *weights)
        u_b, rho_b = decode(psums, dec1W, r1(dec1b), dec2W, r1(dec2b),
                            tau, r1(tau_max), lambda_min_0.reshape(1, 1),
                            r1(K))
        us.append(u_b)
        if rho is None:
            rho = rho_b.reshape(-1)
    u = jnp.concatenate(us, axis=0)
    return (u, rho, K)
